# Initial kernel scaffold; baseline (speedup 1.0000x reference)
import jax
import jax.numpy as jnp
from jax.experimental import pallas as pl


def kernel(sdf, cloth_meshes_unposed, smpl_cloth_idx, smpl_cloth_valid, cloth_idx, sdf_thresh, dist_thresh, v_template):
    raise NotImplementedError("write your pallas kernel here")


# speedup vs baseline: 1.2437x; 1.2437x over previous
import functools

import jax
import jax.numpy as jnp
from jax import lax
from jax.experimental import pallas as pl
from jax.experimental.pallas import tpu as pltpu
from jax.experimental.pallas import tpu_sc as plsc

MIN_T2 = 0.02 * 0.02
BIG2 = 9999.0 * 9999.0

NS_PAD = 6912
C_TILE = 128
S_CHUNK = 768
N_CTILES = 8192 // C_TILE


def _dist_kernel(cloth_ref, smplt_ref, pen_ref, m_ref, idx_ref):
    bid = pl.program_id(0)
    c3 = cloth_ref[0]
    cx = c3[:, 0:1]
    cy = c3[:, 1:2]
    cz = c3[:, 2:3]
    lane = lax.broadcasted_iota(jnp.int32, (1, S_CHUNK), 1)

    def body(k, carry):
        m_run, i_run = carry
        off = k * S_CHUNK
        sx = smplt_ref[0, 0:1, pl.ds(off, S_CHUNK)]
        sy = smplt_ref[0, 1:2, pl.ds(off, S_CHUNK)]
        sz = smplt_ref[0, 2:3, pl.ds(off, S_CHUNK)]
        pen = pen_ref[0, 0:1, pl.ds(off, S_CHUNK)]
        dx = cx - sx
        dy = cy - sy
        dz = cz - sz
        d2 = dx * dx + dy * dy + dz * dz + pen
        d2 = jnp.where(d2 < MIN_T2, BIG2, d2)
        upd = d2 < m_run
        m_run = jnp.where(upd, d2, m_run)
        i_run = jnp.where(upd, off + lane, i_run)
        return m_run, i_run

    m0 = jnp.full((C_TILE, S_CHUNK), jnp.inf, jnp.float32)
    i0 = jnp.zeros((C_TILE, S_CHUNK), jnp.int32)
    m_run, i_run = lax.fori_loop(0, NS_PAD // S_CHUNK, body, (m0, i0))

    m = jnp.min(m_run, axis=1, keepdims=True)
    big_i = jnp.int32(2 ** 30)
    isel = jnp.min(jnp.where(m_run == m, i_run, big_i), axis=1, keepdims=True)
    m_ref[0, 0] = m
    idx_ref[0, 0] = isel + bid * NS_PAD


def _nearest(smplt, cloth, pen):
    B = cloth.shape[0]
    grid = (B, N_CTILES)
    out_shape = [
        jax.ShapeDtypeStruct((B, N_CTILES, C_TILE, 1), jnp.float32),
        jax.ShapeDtypeStruct((B, N_CTILES, C_TILE, 1), jnp.int32),
    ]
    m, idx = pl.pallas_call(
        _dist_kernel,
        grid=grid,
        in_specs=[
            pl.BlockSpec((1, C_TILE, 3), lambda b, c: (b, c, 0)),
            pl.BlockSpec((1, 3, NS_PAD), lambda b, c: (b, 0, 0)),
            pl.BlockSpec((1, 1, NS_PAD), lambda b, c: (b, 0, 0)),
        ],
        out_specs=[
            pl.BlockSpec((1, 1, C_TILE, 1), lambda b, c: (b, c, 0, 0)),
            pl.BlockSpec((1, 1, C_TILE, 1), lambda b, c: (b, c, 0, 0)),
        ],
        out_shape=out_shape,
        compiler_params=pltpu.CompilerParams(
            dimension_semantics=("parallel", "parallel"),
        ),
    )(cloth, smplt, pen)
    return m.reshape(B, -1), idx.reshape(B, -1)


N_IROWS = 8192 // 128


def _sc_loss_kernel(m_hbm, idx_hbm, sdf_hbm, lab_hbm, cvec_hbm, dt_hbm, st_hbm,
                    out_hbm, idx_v, gath_v, m_v, sdf_v, sc_v, sem):
    NC_SC = 8192
    cid = lax.axis_index("c")
    sid = lax.axis_index("s")
    wid = cid * 16 + sid

    @pl.when(wid < 8)
    def _():
        pltpu.sync_copy(idx_hbm.at[pl.ds(wid * N_IROWS, N_IROWS)], idx_v)
        pltpu.sync_copy(m_hbm.at[pl.ds(wid * NC_SC, NC_SC)], m_v)
        pltpu.sync_copy(sdf_hbm.at[pl.ds(wid * NC_SC, NC_SC)], sdf_v)
        pltpu.sync_copy(cvec_hbm, sc_v.at[0])
        pltpu.sync_copy(dt_hbm, sc_v.at[1])
        pltpu.sync_copy(st_hbm, sc_v.at[2])

        copies = [
            pltpu.async_copy(lab_hbm.at[idx_v.at[j]], gath_v.at[j], sem)
            for j in range(N_IROWS)
        ]
        for c in copies:
            c.wait()

        cvec = sc_v[0]
        dt = sc_v[1]
        st = sc_v[2]
        dt2 = dt * dt

        def body(j, carry):
            acc, cnt = carry
            for k in range(8):
                lab = gath_v[j, pl.ds(k * 16, 16)]
                sl = pl.ds(j * 128 + k * 16, 16)
                mf = jnp.where(lab == cvec, 1.0, 0.0).astype(jnp.float32)
                d2 = m_v[sl]
                s = sdf_v[sl]
                nf = jnp.where(d2 < dt2, 1.0, 0.0).astype(jnp.float32)
                lp = jnp.abs(s) * mf
                ln = jnp.abs(s - st) * (1.0 - mf)
                acc = acc + (lp + ln) * nf
                cnt = cnt + mf
            return acc, cnt

        z = jnp.zeros((16,), jnp.float32)
        acc, cnt = lax.fori_loop(0, N_IROWS, body, (z, z))
        sc_v[4] = acc
        sc_v[5] = cnt
        pltpu.sync_copy(sc_v.at[pl.ds(4, 2)], out_hbm.at[wid])


def _sc_loss(m, idx, sdf, lab, cvec, dtv, stv):
    B = sdf.shape[0]
    mesh = plsc.VectorSubcoreMesh(core_axis_name="c", subcore_axis_name="s")
    fn = functools.partial(
        pl.kernel,
        mesh=mesh,
        out_type=jax.ShapeDtypeStruct((B, 2, 16), jnp.float32),
        scratch_types=[
            pltpu.VMEM((N_IROWS, 128), jnp.int32),
            pltpu.VMEM((N_IROWS, 128), jnp.float32),
            pltpu.VMEM((8192,), jnp.float32),
            pltpu.VMEM((8192,), jnp.float32),
            pltpu.VMEM((6, 16), jnp.float32),
            pltpu.SemaphoreType.DMA,
        ],
    )(_sc_loss_kernel)
    out = fn(m.reshape(-1), idx.reshape(B * N_IROWS, 128), sdf.reshape(-1),
             lab.reshape(-1), cvec, dtv, stv)
    total = out[:, 0, :].sum(axis=1)
    n_in = out[:, 1, :].sum(axis=1)
    return total * (1.0 / 8192.0) * (n_in > 0.0).astype(jnp.float32)


def kernel(sdf, cloth_meshes_unposed, smpl_cloth_idx, smpl_cloth_valid,
           cloth_idx, sdf_thresh, dist_thresh, v_template):
    B, Nc, _ = cloth_meshes_unposed.shape
    Ns = v_template.shape[1]
    pad = NS_PAD - Ns

    smplt = jnp.swapaxes(v_template, 1, 2)
    smplt = jnp.pad(smplt, ((0, 0), (0, 0), (0, pad)))
    validp = jnp.pad(smpl_cloth_valid, ((0, 0), (0, pad)))
    pen = jnp.where(validp > 0, jnp.float32(0.0),
                    jnp.float32(jnp.inf)).reshape(B, 1, NS_PAD)

    m, idx = _nearest(smplt, cloth_meshes_unposed, pen)

    lab = jnp.pad(smpl_cloth_idx, ((0, 0), (0, pad))).astype(jnp.float32)
    cvec = jnp.broadcast_to(cloth_idx[0].astype(jnp.float32), (16,))
    dtv = jnp.broadcast_to(dist_thresh.astype(jnp.float32), (16,))
    stv = jnp.broadcast_to(sdf_thresh.astype(jnp.float32), (16,))

    return _sc_loss(m, idx, sdf, lab, cvec, dtv, stv)
